# baseline (speedup 1.0000x reference)
import functools

import jax
import jax.numpy as jnp
from jax import lax
from jax.experimental import pallas as pl
from jax.experimental.pallas import tpu as pltpu
from jax.experimental.pallas import tpu_sc as plsc

N = 10000
E = 160000
D_IN = 256
H = 512
CW = 128
NCHUNK = H // CW
NSC = 2
NTILE = 16
E_PAD = 163840
NB16 = E_PAD // 16 // 128
NB32 = E_PAD // 32 // 128
PAD_ROW = N
NP = 10240
ACC_ROWS = NP
RPT = NP // NTILE
NBUF = 2

def _mesh():
    return plsc.VectorSubcoreMesh(core_axis_name="c", subcore_axis_name="s")


def _sc_degree(dst32, ones, zeros):
    @functools.partial(
        pl.kernel,
        out_type=jax.ShapeDtypeStruct((NSC, ACC_ROWS, 128), jnp.float32),
        mesh=_mesh(),
        scratch_types=[
            pltpu.VMEM((NB32, 128), jnp.int32),
            pltpu.VMEM((128, 128), jnp.float32),
            pltpu.VMEM_SHARED((ACC_ROWS, 128), jnp.float32),
        ],
    )
    def k(dst_hbm, ones_hbm, zeros_hbm, out_hbm, dst_v, ones_v, hist):
        ci = lax.axis_index("c")
        ti = lax.axis_index("s")
        pltpu.sync_copy(dst_hbm.at[ci * NTILE + ti], dst_v)
        pltpu.sync_copy(ones_hbm, ones_v)
        pltpu.sync_copy(
            zeros_hbm.at[pl.ds(ti * RPT, RPT)],
            hist.at[pl.ds(ti * RPT, RPT)],
        )
        plsc.subcore_barrier()

        @pl.loop(0, NB32)
        def _(j):
            pltpu.sync_copy(ones_v, hist.at[dst_v.at[j]], add=True)

        plsc.subcore_barrier()
        pltpu.sync_copy(
            hist.at[pl.ds(ti * RPT, RPT)],
            out_hbm.at[ci].at[pl.ds(ti * RPT, RPT)],
        )

    return k(dst32, ones, zeros)


def _sc_spmm(y4, packed16):
    @functools.partial(
        pl.kernel,
        out_type=jax.ShapeDtypeStruct((NCHUNK, NP, CW), jnp.float32),
        mesh=_mesh(),
        scratch_types=[
            pltpu.VMEM((NB16, 128), jnp.int32),
            pltpu.VMEM((8, 128), jnp.int32),
            pltpu.VMEM((8, 128), jnp.int32),
        ]
        + [pltpu.VMEM((128, CW), jnp.float32) for _ in range(NBUF)]
        + [pltpu.SemaphoreType.DMA for _ in range(2 * NBUF)]
        + [pltpu.VMEM_SHARED((ACC_ROWS, CW), jnp.float32)],
    )
    def k(y_hbm, pk_hbm, out_hbm, pk_v, sidx, didx, *rest):
        bufs = rest[:NBUF]
        sems = rest[NBUF : 2 * NBUF]
        ssems = rest[2 * NBUF : 3 * NBUF]
        acc = rest[3 * NBUF]
        ci = lax.axis_index("c")
        ti = lax.axis_index("s")
        pltpu.sync_copy(pk_hbm.at[ti], pk_v)

        def unpack_src(j, r):
            for l in range(8):
                v = pk_v[j, pl.ds(l * 16, 16)]
                sidx[r, pl.ds(l * 16, 16)] = v & 0x3FFF

        def unpack_dst(j, r):
            for l in range(8):
                v = pk_v[j, pl.ds(l * 16, 16)]
                didx[r, pl.ds(l * 16, 16)] = lax.shift_right_logical(v, 14)

        for kk in range(NCHUNK // NSC):
            c = ci * (NCHUNK // NSC) + kk
            yc = y_hbm.at[c]
            pltpu.sync_copy(
                yc.at[pl.ds(ti * RPT, RPT)],
                acc.at[pl.ds(ti * RPT, RPT)],
            )
            plsc.subcore_barrier()

            for r in range(NBUF):
                unpack_src(r, r)
                pltpu.async_copy(yc.at[sidx.at[r]], bufs[r], sems[r])

            @pl.loop(0, NB16 - NBUF, step=NBUF)
            def _(j0):
                handles = []
                for r in range(NBUF):
                    j = j0 + r
                    unpack_dst(j, r)
                    pltpu.make_async_copy(
                        yc.at[sidx.at[r]], bufs[r], sems[r]
                    ).wait()
                    handles.append(
                        pltpu.async_copy(
                            bufs[r], acc.at[didx.at[r]], ssems[r], add=True
                        )
                    )
                for r in range(NBUF):
                    unpack_src(j0 + r + NBUF, r)
                for r in range(NBUF):
                    handles[r].wait()
                    pltpu.async_copy(yc.at[sidx.at[r]], bufs[r], sems[r])

            for r in range(NBUF):
                j = NB16 - NBUF + r
                unpack_dst(j, r)
                pltpu.make_async_copy(
                    yc.at[sidx.at[r]], bufs[r], sems[r]
                ).wait()
                pltpu.async_copy(
                    bufs[r], acc.at[didx.at[r]], ssems[r], add=True
                ).wait()

            plsc.subcore_barrier()
            pltpu.sync_copy(
                acc.at[pl.ds(ti * RPT, RPT)],
                out_hbm.at[c].at[pl.ds(ti * RPT, RPT)],
            )
            plsc.subcore_barrier()

    return k(y4, packed16)


BN_MM = 2000
BN = 2000


def _tc_matmul(x, W, b):
    def body(x_ref, w_ref, b_ref, o_ref):
        o_ref[...] = (
            jnp.dot(x_ref[...], w_ref[...], preferred_element_type=jnp.float32)
            + b_ref[...]
        )

    return pl.pallas_call(
        body,
        grid=(N // BN_MM,),
        in_specs=[
            pl.BlockSpec((BN_MM, D_IN), lambda i: (i, 0)),
            pl.BlockSpec((D_IN, H), lambda i: (0, 0)),
            pl.BlockSpec((1, H), lambda i: (0, 0)),
        ],
        out_specs=pl.BlockSpec((BN_MM, H), lambda i: (i, 0)),
        out_shape=jax.ShapeDtypeStruct((N, H), jnp.float32),
    )(x, W, b.reshape(1, H))


def _tc_prep(h0, degp):
    def body(h_ref, d_ref, norm_ref, y_ref):
        deg = d_ref[0, :, 0] + d_ref[1, :, 0]
        norm = lax.rsqrt(1.0 + deg).reshape(BN, 1)
        norm_ref[...] = norm
        y = h_ref[...] * norm
        for c in range(NCHUNK):
            y_ref[c] = y[:, c * CW : (c + 1) * CW]

    return pl.pallas_call(
        body,
        grid=(N // BN,),
        in_specs=[
            pl.BlockSpec((BN, H), lambda i: (i, 0)),
            pl.BlockSpec((2, BN, 128), lambda i: (0, i, 0)),
        ],
        out_specs=[
            pl.BlockSpec((BN, 1), lambda i: (i, 0)),
            pl.BlockSpec((NCHUNK, BN, CW), lambda i: (0, i, 0)),
        ],
        out_shape=[
            jax.ShapeDtypeStruct((N, 1), jnp.float32),
            jax.ShapeDtypeStruct((NCHUNK, NP, CW), jnp.float32),
        ],
    )(h0, degp)


def _tc_layer(agg4, norm, ori, g, bb):
    def body(a_ref, n_ref, o_ref, g_ref, b_ref, y_ref):
        t = jnp.concatenate([a_ref[c] for c in range(NCHUNK)], axis=-1)
        nrm = n_ref[...]
        t = t * nrm + o_ref[...]
        mu = jnp.mean(t, axis=-1, keepdims=True)
        var = jnp.mean((t - mu) ** 2, axis=-1, keepdims=True)
        u = (t - mu) * lax.rsqrt(var + 1e-5) * g_ref[...] + b_ref[...]
        u = jnp.maximum(u, 0.0) * nrm
        for c in range(NCHUNK):
            y_ref[c] = u[:, c * CW : (c + 1) * CW]

    return pl.pallas_call(
        body,
        grid=(N // BN,),
        in_specs=[
            pl.BlockSpec((NCHUNK, BN, CW), lambda i: (0, i, 0)),
            pl.BlockSpec((BN, 1), lambda i: (i, 0)),
            pl.BlockSpec((BN, H), lambda i: (i, 0)),
            pl.BlockSpec((1, H), lambda i: (0, 0)),
            pl.BlockSpec((1, H), lambda i: (0, 0)),
        ],
        out_specs=pl.BlockSpec((NCHUNK, BN, CW), lambda i: (0, i, 0)),
        out_shape=jax.ShapeDtypeStruct((NCHUNK, NP, CW), jnp.float32),
    )(agg4, norm, ori, g.reshape(1, H), bb.reshape(1, H))


def _tc_final(agg4, norm):
    def body(a_ref, n_ref, o_ref):
        t = jnp.concatenate([a_ref[c] for c in range(NCHUNK)], axis=-1)
        o_ref[...] = t * n_ref[...]

    return pl.pallas_call(
        body,
        grid=(N // BN,),
        in_specs=[
            pl.BlockSpec((NCHUNK, BN, CW), lambda i: (0, i, 0)),
            pl.BlockSpec((BN, 1), lambda i: (i, 0)),
        ],
        out_specs=pl.BlockSpec((BN, H), lambda i: (i, 0)),
        out_shape=jax.ShapeDtypeStruct((N, H), jnp.float32),
    )(agg4, norm)


def kernel(x, edge_index, W, b, ln1_g, ln1_b, ln2_g, ln2_b):
    dst = edge_index[0]
    src = edge_index[1]
    pad = E_PAD - E
    dst_p = jnp.concatenate([dst, jnp.full((pad,), PAD_ROW, jnp.int32)])
    src_p = jnp.concatenate([src, jnp.zeros((pad,), jnp.int32)])
    dst32 = dst_p.reshape(NSC * NTILE, NB32, 128)
    packed16 = (dst_p * 16384 + src_p).reshape(NTILE, NB16, 128)
    ones = jnp.ones((128, 128), jnp.float32)
    zeros = jnp.zeros((ACC_ROWS, 128), jnp.float32)

    degp = _sc_degree(dst32, ones, zeros)
    h0 = _tc_matmul(x, W, b)
    norm, y = _tc_prep(h0, degp)
    out = None
    for i in range(3):
        agg4 = _sc_spmm(y, packed16)
        if i < 2:
            g, bb = (ln1_g, ln1_b) if i == 0 else (ln2_g, ln2_b)
            y = _tc_layer(agg4, norm, h0, g, bb)
        else:
            out = _tc_final(agg4, norm)
    return out


# speedup vs baseline: 1.8207x; 1.8207x over previous
import functools

import jax
import jax.numpy as jnp
from jax import lax
from jax.experimental import pallas as pl
from jax.experimental.pallas import tpu as pltpu
from jax.experimental.pallas import tpu_sc as plsc

N = 10000
E = 160000
D_IN = 256
H = 512
CW = 128
NCHUNK = H // CW
NSC = 2
NTILE = 16
E_PAD = 163840
NB16 = E_PAD // 16 // 128
NB32 = E_PAD // 32 // 128
PAD_ROW = N
NP = 10240
ACC_ROWS = NP
RPT = NP // NTILE
BKS = 120
NBS = 84
E_PAD_SP = NTILE * NBS * BKS

def _mesh():
    return plsc.VectorSubcoreMesh(core_axis_name="c", subcore_axis_name="s")


def _sc_degree(dst32, ones, zeros):
    @functools.partial(
        pl.kernel,
        out_type=jax.ShapeDtypeStruct((NSC, ACC_ROWS, 128), jnp.float32),
        mesh=_mesh(),
        scratch_types=[
            pltpu.VMEM((NB32, 128), jnp.int32),
            pltpu.VMEM((128, 128), jnp.float32),
            pltpu.VMEM_SHARED((ACC_ROWS, 128), jnp.float32),
        ],
    )
    def k(dst_hbm, ones_hbm, zeros_hbm, out_hbm, dst_v, ones_v, hist):
        ci = lax.axis_index("c")
        ti = lax.axis_index("s")
        pltpu.sync_copy(dst_hbm.at[ci * NTILE + ti], dst_v)
        pltpu.sync_copy(ones_hbm, ones_v)
        pltpu.sync_copy(
            zeros_hbm.at[pl.ds(ti * RPT, RPT)],
            hist.at[pl.ds(ti * RPT, RPT)],
        )
        plsc.subcore_barrier()

        @pl.loop(0, NB32)
        def _(j):
            pltpu.sync_copy(ones_v, hist.at[dst_v.at[j]], add=True)

        plsc.subcore_barrier()
        pltpu.sync_copy(
            hist.at[pl.ds(ti * RPT, RPT)],
            out_hbm.at[ci].at[pl.ds(ti * RPT, RPT)],
        )

    return k(dst32, ones, zeros)


def _sc_spmm(y4, src4, dst4):
    R = 3

    @functools.partial(
        pl.kernel,
        out_type=jax.ShapeDtypeStruct((NCHUNK, NP, CW), jnp.float32),
        mesh=_mesh(),
        scratch_types=[
            pltpu.VMEM((8, BKS), jnp.int32),
            pltpu.VMEM((8, BKS), jnp.int32),
        ]
        + [pltpu.VMEM((BKS, CW), jnp.float32) for _ in range(R)]
        + [pltpu.SemaphoreType.DMA for _ in range(4 * R)]
        + [pltpu.VMEM_SHARED((ACC_ROWS, CW), jnp.float32)],
    )
    def k(y_hbm, src_hbm, dst_hbm, out_hbm, sidx, didx, *rest):
        bufs = rest[:R]
        gsem = rest[R : 2 * R]
        ssem = rest[2 * R : 3 * R]
        issem = rest[3 * R : 4 * R]
        idsem = rest[4 * R : 5 * R]
        acc = rest[5 * R]
        ci = lax.axis_index("c")
        ti = lax.axis_index("s")
        src_t = src_hbm.at[ti]
        dst_t = dst_hbm.at[ti]

        def load_sidx(j, r):
            pltpu.async_copy(src_t.at[j], sidx.at[pl.ds(r, 1)], issem[r])

        def load_didx(j, r):
            pltpu.async_copy(dst_t.at[j], didx.at[pl.ds(r, 1)], idsem[r])

        def wait_sidx(r):
            pltpu.make_async_copy(
                src_t.at[0], sidx.at[pl.ds(r, 1)], issem[r]
            ).wait()

        def wait_didx(r):
            pltpu.make_async_copy(
                dst_t.at[0], didx.at[pl.ds(r, 1)], idsem[r]
            ).wait()

        def gather(j, r, yc):
            pltpu.async_copy(yc.at[sidx.at[r]], bufs[r], gsem[r])

        def wait_gather(r, yc):
            pltpu.make_async_copy(
                yc.at[sidx.at[r]], bufs[r], gsem[r]
            ).wait()

        def scatter(r):
            pltpu.async_copy(bufs[r], acc.at[didx.at[r]], ssem[r], add=True)

        def wait_scatter(r):
            pltpu.make_async_copy(bufs[r], acc.at[didx.at[r]], ssem[r]).wait()

        for kk in range(NCHUNK // NSC):
            c = ci * (NCHUNK // NSC) + kk
            yc = y_hbm.at[c]
            pltpu.sync_copy(
                yc.at[pl.ds(ti * RPT, RPT)],
                acc.at[pl.ds(ti * RPT, RPT)],
            )
            plsc.subcore_barrier()

            for r in range(R):
                load_sidx(r, r)
            load_didx(0, 0)
            load_didx(1, 1)
            wait_sidx(0)
            gather(0, 0, yc)
            wait_sidx(1)
            gather(1, 1, yc)

            wait_gather(0, yc)
            load_sidx(3, 0)
            wait_didx(0)
            scatter(0)
            load_didx(2, 2)
            wait_sidx(2)
            gather(2, 2, yc)

            wait_gather(1, yc)
            load_sidx(4, 1)
            wait_didx(1)
            scatter(1)
            wait_scatter(0)
            load_didx(3, 0)
            wait_sidx(0)
            gather(3, 0, yc)

            wait_gather(2, yc)
            load_sidx(5, 2)
            wait_didx(2)
            scatter(2)
            wait_scatter(1)
            load_didx(4, 1)
            wait_sidx(1)
            gather(4, 1, yc)

            @pl.loop(3, NBS - 3, step=R)
            def _(j0):
                for u in range(R):
                    j = j0 + u
                    r = u % R
                    r2 = (u + 2) % R
                    wait_gather(r, yc)
                    load_sidx(j + R, r)
                    wait_didx(r)
                    scatter(r)
                    wait_scatter(r2)
                    load_didx(j + 2, r2)
                    wait_sidx(r2)
                    gather(j + 2, r2, yc)

            j = NBS - 3
            r, r2 = j % R, (j - 1) % R
            wait_gather(r, yc)
            wait_didx(r)
            scatter(r)
            wait_scatter(r2)
            load_didx(j + 2, r2)
            wait_sidx(r2)
            gather(j + 2, r2, yc)

            j = NBS - 2
            r, r2 = j % R, (j - 1) % R
            wait_gather(r, yc)
            wait_didx(r)
            scatter(r)
            wait_scatter(r2)

            j = NBS - 1
            r, r2 = j % R, (j - 1) % R
            wait_gather(r, yc)
            wait_didx(r)
            scatter(r)
            wait_scatter(r2)
            wait_scatter(r)

            plsc.subcore_barrier()
            pltpu.sync_copy(
                acc.at[pl.ds(ti * RPT, RPT)],
                out_hbm.at[c].at[pl.ds(ti * RPT, RPT)],
            )
            plsc.subcore_barrier()

    return k(y4, src4, dst4)


BN_MM = 2000
BN = 2000


def _tc_matmul(x, W, b):
    def body(x_ref, w_ref, b_ref, o_ref):
        o_ref[...] = (
            jnp.dot(x_ref[...], w_ref[...], preferred_element_type=jnp.float32)
            + b_ref[...]
        )

    return pl.pallas_call(
        body,
        grid=(N // BN_MM,),
        in_specs=[
            pl.BlockSpec((BN_MM, D_IN), lambda i: (i, 0)),
            pl.BlockSpec((D_IN, H), lambda i: (0, 0)),
            pl.BlockSpec((1, H), lambda i: (0, 0)),
        ],
        out_specs=pl.BlockSpec((BN_MM, H), lambda i: (i, 0)),
        out_shape=jax.ShapeDtypeStruct((N, H), jnp.float32),
    )(x, W, b.reshape(1, H))


def _tc_prep(h0, degp):
    def body(h_ref, d_ref, norm_ref, y_ref):
        deg = d_ref[0, :, 0] + d_ref[1, :, 0]
        norm = lax.rsqrt(1.0 + deg).reshape(BN, 1)
        norm_ref[...] = norm
        y = h_ref[...] * norm
        for c in range(NCHUNK):
            y_ref[c] = y[:, c * CW : (c + 1) * CW]

    return pl.pallas_call(
        body,
        grid=(N // BN,),
        in_specs=[
            pl.BlockSpec((BN, H), lambda i: (i, 0)),
            pl.BlockSpec((2, BN, 128), lambda i: (0, i, 0)),
        ],
        out_specs=[
            pl.BlockSpec((BN, 1), lambda i: (i, 0)),
            pl.BlockSpec((NCHUNK, BN, CW), lambda i: (0, i, 0)),
        ],
        out_shape=[
            jax.ShapeDtypeStruct((N, 1), jnp.float32),
            jax.ShapeDtypeStruct((NCHUNK, NP, CW), jnp.float32),
        ],
    )(h0, degp)


def _tc_layer(agg4, norm, ori, g, bb):
    def body(a_ref, n_ref, o_ref, g_ref, b_ref, y_ref):
        t = jnp.concatenate([a_ref[c] for c in range(NCHUNK)], axis=-1)
        nrm = n_ref[...]
        t = t * nrm + o_ref[...]
        mu = jnp.mean(t, axis=-1, keepdims=True)
        var = jnp.mean((t - mu) ** 2, axis=-1, keepdims=True)
        u = (t - mu) * lax.rsqrt(var + 1e-5) * g_ref[...] + b_ref[...]
        u = jnp.maximum(u, 0.0) * nrm
        for c in range(NCHUNK):
            y_ref[c] = u[:, c * CW : (c + 1) * CW]

    return pl.pallas_call(
        body,
        grid=(N // BN,),
        in_specs=[
            pl.BlockSpec((NCHUNK, BN, CW), lambda i: (0, i, 0)),
            pl.BlockSpec((BN, 1), lambda i: (i, 0)),
            pl.BlockSpec((BN, H), lambda i: (i, 0)),
            pl.BlockSpec((1, H), lambda i: (0, 0)),
            pl.BlockSpec((1, H), lambda i: (0, 0)),
        ],
        out_specs=pl.BlockSpec((NCHUNK, BN, CW), lambda i: (0, i, 0)),
        out_shape=jax.ShapeDtypeStruct((NCHUNK, NP, CW), jnp.float32),
    )(agg4, norm, ori, g.reshape(1, H), bb.reshape(1, H))


def _tc_final(agg4, norm):
    def body(a_ref, n_ref, o_ref):
        t = jnp.concatenate([a_ref[c] for c in range(NCHUNK)], axis=-1)
        o_ref[...] = t * n_ref[...]

    return pl.pallas_call(
        body,
        grid=(N // BN,),
        in_specs=[
            pl.BlockSpec((NCHUNK, BN, CW), lambda i: (0, i, 0)),
            pl.BlockSpec((BN, 1), lambda i: (i, 0)),
        ],
        out_specs=pl.BlockSpec((BN, H), lambda i: (i, 0)),
        out_shape=jax.ShapeDtypeStruct((N, H), jnp.float32),
    )(agg4, norm)


def kernel(x, edge_index, W, b, ln1_g, ln1_b, ln2_g, ln2_b):
    dst = edge_index[0]
    src = edge_index[1]
    pad = E_PAD - E
    dst_p = jnp.concatenate([dst, jnp.full((pad,), PAD_ROW, jnp.int32)])
    src_p = jnp.concatenate([src, jnp.zeros((pad,), jnp.int32)])
    dst32 = dst_p.reshape(NSC * NTILE, NB32, 128)
    pad_sp = E_PAD_SP - E
    dst_sp = jnp.concatenate([dst, jnp.full((pad_sp,), PAD_ROW, jnp.int32)])
    src_sp = jnp.concatenate([src, jnp.zeros((pad_sp,), jnp.int32)])
    src4 = src_sp.reshape(NTILE, NBS, 1, BKS)
    dst4 = dst_sp.reshape(NTILE, NBS, 1, BKS)
    ones = jnp.ones((128, 128), jnp.float32)
    zeros = jnp.zeros((ACC_ROWS, 128), jnp.float32)

    degp = _sc_degree(dst32, ones, zeros)
    h0 = _tc_matmul(x, W, b)
    norm, y = _tc_prep(h0, degp)
    out = None
    for i in range(3):
        agg4 = _sc_spmm(y, src4, dst4)
        if i < 2:
            g, bb = (ln1_g, ln1_b) if i == 0 else (ln2_g, ln2_b)
            y = _tc_layer(agg4, norm, h0, g, bb)
        else:
            out = _tc_final(agg4, norm)
    return out
